# baseline (speedup 1.0000x reference)
import jax
import jax.numpy as jnp
from jax import lax
from jax.experimental import pallas as pl
from jax.experimental.pallas import tpu as pltpu
from jax.experimental.pallas import tpu_sc as plsc
import functools

B = 16384
L = 50
D = 64
NW = 32
BW = B // NW
CB = 4
CI = CB * L
NCH = BW // CB
NBUF = 4

V = 1000000
TBLK = 8192
TGRID = (V + TBLK - 1) // TBLK
V_PAD = TGRID * TBLK
H = TBLK // 2


Q = TBLK // 4


def _transpose_body(tt_ref, out_ref):
    x = tt_ref[...]
    u = lax.bitcast_convert_type(x, jnp.uint32)
    r = u + jnp.uint32(0x8000)
    hw = r >> 16
    w = hw[0:D // 2, :] | (hw[D // 2:D, :] << 16)
    wi = lax.bitcast_convert_type(w, jnp.int32)
    for q in range(4):
        out_ref[:, 32 * q:32 * (q + 1)] = wi[:, q * Q:(q + 1) * Q].T


_table_to_rowmajor = pl.pallas_call(
    _transpose_body,
    grid=(TGRID,),
    in_specs=[pl.BlockSpec((D, TBLK), lambda j: (0, j))],
    out_specs=pl.BlockSpec((Q, 128), lambda j: (j, 0)),
    out_shape=jax.ShapeDtypeStruct((V_PAD // 4, 128), jnp.int32),
)


def _remap_indices(i):
    q = (i & (TBLK - 1)) >> 11
    a = i & (Q - 1)
    return (i & ~(TBLK - 1)) + (a << 2) + q


def _lane_splat(vec, idx):
    return lax.gather(
        vec, idx,
        dimension_numbers=lax.GatherDimensionNumbers(
            offset_dims=(), collapsed_slice_dims=(0,), start_index_map=(0,)),
        slice_sizes=(1,),
        mode=lax.GatherScatterMode.PROMISE_IN_BOUNDS)


_mesh = plsc.VectorSubcoreMesh(
    core_axis_name="c", subcore_axis_name="s", num_cores=2, num_subcores=16
)


@functools.partial(
    pl.kernel,
    out_type=jax.ShapeDtypeStruct((NW, BW, D), jnp.float32),
    mesh=_mesh,
    compiler_params=pltpu.CompilerParams(use_tc_tiling_on_sc=False),
    scratch_types=[
        pltpu.VMEM((NCH, CI), jnp.int32),
        pltpu.VMEM((NCH, CB * 64), jnp.float32),
        pltpu.VMEM((NBUF, CI, D // 2), jnp.int32),
        pltpu.VMEM((BW, D), jnp.float32),
    ] + [pltpu.SemaphoreType.DMA] * NBUF,
)
def _embedding_bag_sc(table_hbm, idx_hbm, w_hbm, out_hbm,
                      idx_v, w_v, rows_v, out_v, *sems):
    wid = lax.axis_index("s") * 2 + lax.axis_index("c")
    pltpu.sync_copy(idx_hbm.at[wid], idx_v)
    pltpu.sync_copy(w_hbm.at[wid], w_v)

    for b in range(NBUF - 1):
        pltpu.async_copy(table_hbm.at[idx_v.at[b]], rows_v.at[b], sems[b])

    lane_idx = [jnp.full((16, 1), m, jnp.int32) for m in range(16)]

    def compute_chunk(j, b):
        for k in range(CB):
            wv = [w_v[j, pl.ds(k * 64 + g * 16, 16)] for g in range(4)]
            acc = [jnp.zeros((16,), jnp.float32) for _ in range(D // 16)]
            for l in range(L):
                r = k * L + l
                wsp = _lane_splat(wv[l // 16], lane_idx[l % 16])
                for g in range(2):
                    wrd = rows_v[b, r, pl.ds(g * 16, 16)]
                    fa = lax.bitcast_convert_type(wrd << 16, jnp.float32)
                    fb = lax.bitcast_convert_type(wrd, jnp.float32)
                    acc[g] = acc[g] + wsp * fa
                    acc[g + 2] = acc[g + 2] + wsp * fb
            bag = j * CB + k
            for c in range(D // 16):
                out_v[bag, pl.ds(c * 16, 16)] = acc[c]

    @pl.loop(0, NCH // NBUF)
    def _ring(i):
        j0 = i * NBUF
        for b in range(NBUF):
            j = j0 + b
            nxt = j + NBUF - 1
            pb = (b - 1) % NBUF

            @pl.when(nxt < NCH)
            def _():
                pltpu.async_copy(
                    table_hbm.at[idx_v.at[nxt]], rows_v.at[pb], sems[pb])

            pltpu.make_async_copy(
                table_hbm.at[idx_v.at[j]], rows_v.at[b], sems[b]).wait()
            compute_chunk(j, b)

    pltpu.sync_copy(out_v, out_hbm.at[wid])


def kernel(indices, weights, embeddings):
    idx3 = _remap_indices(indices.astype(jnp.int32)).reshape(NW, NCH, CI)
    w_pad = jnp.pad(weights, ((0, 0), (0, 64 - L)))
    w3 = w_pad.reshape(NW, NCH, CB * 64)
    table2 = _table_to_rowmajor(embeddings.T)
    out = _embedding_bag_sc(table2.reshape(V_PAD, D // 2), idx3, w3)
    return out.reshape(B, D)


# speedup vs baseline: 1.0667x; 1.0667x over previous
import jax
import jax.numpy as jnp
from jax import lax
from jax.experimental import pallas as pl
from jax.experimental.pallas import tpu as pltpu
from jax.experimental.pallas import tpu_sc as plsc
import functools

B = 16384
L = 50
D = 64
NW = 32
BW = B // NW
CB = 2
CI = CB * L
NCH = BW // CB
NBUF = 4

V = 1000000
TBLK = 16384
TGRID = (V + TBLK - 1) // TBLK
V_PAD = TGRID * TBLK
H = TBLK // 2


Q = TBLK // 4


def _transpose_body(tt_ref, out_ref):
    x = tt_ref[...]
    u = lax.bitcast_convert_type(x, jnp.uint32)
    r = u + jnp.uint32(0x8000)
    hw = r >> 16
    w = hw[0:D // 2, :] | (hw[D // 2:D, :] << 16)
    wi = lax.bitcast_convert_type(w, jnp.int32)
    out_ref[...] = jnp.concatenate(
        [wi[:, q * Q:(q + 1) * Q].T for q in range(4)], axis=1)


_table_to_rowmajor = pl.pallas_call(
    _transpose_body,
    grid=(TGRID,),
    in_specs=[pl.BlockSpec((D, TBLK), lambda j: (0, j))],
    out_specs=pl.BlockSpec((Q, 128), lambda j: (j, 0)),
    out_shape=jax.ShapeDtypeStruct((V_PAD // 4, 128), jnp.int32),
)


def _remap_indices(i):
    q = (i & (TBLK - 1)) >> 12
    a = i & (Q - 1)
    return (i & ~(TBLK - 1)) + (a << 2) + q


def _lane_splat(vec, idx):
    return lax.gather(
        vec, idx,
        dimension_numbers=lax.GatherDimensionNumbers(
            offset_dims=(), collapsed_slice_dims=(0,), start_index_map=(0,)),
        slice_sizes=(1,),
        mode=lax.GatherScatterMode.PROMISE_IN_BOUNDS)


_mesh = plsc.VectorSubcoreMesh(
    core_axis_name="c", subcore_axis_name="s", num_cores=2, num_subcores=16
)


@functools.partial(
    pl.kernel,
    out_type=jax.ShapeDtypeStruct((NW, BW, D), jnp.float32),
    mesh=_mesh,
    compiler_params=pltpu.CompilerParams(use_tc_tiling_on_sc=False),
    scratch_types=[
        pltpu.VMEM((NCH, CI), jnp.int32),
        pltpu.VMEM((NCH, CB * 64), jnp.float32),
        pltpu.VMEM((NBUF, CI, D // 2), jnp.int32),
        pltpu.VMEM((BW, D), jnp.float32),
    ] + [pltpu.SemaphoreType.DMA] * NBUF,
)
def _embedding_bag_sc(table_hbm, idx_hbm, w_hbm, out_hbm,
                      idx_v, w_v, rows_v, out_v, *sems):
    wid = lax.axis_index("s") * 2 + lax.axis_index("c")
    pltpu.sync_copy(idx_hbm.at[wid], idx_v)
    pltpu.sync_copy(w_hbm.at[wid], w_v)

    for b in range(NBUF - 1):
        pltpu.async_copy(table_hbm.at[idx_v.at[b]], rows_v.at[b], sems[b])

    lane_idx = [jnp.full((16, 1), m, jnp.int32) for m in range(16)]

    def compute_chunk(j, b):
        for k in range(CB):
            wv = [w_v[j, pl.ds(k * 64 + g * 16, 16)] for g in range(4)]
            acc = [jnp.zeros((16,), jnp.float32) for _ in range(D // 16)]
            for l in range(L):
                r = k * L + l
                wsp = _lane_splat(wv[l // 16], lane_idx[l % 16])
                for g in range(2):
                    wrd = rows_v[b, r, pl.ds(g * 16, 16)]
                    fa = lax.bitcast_convert_type(wrd << 16, jnp.float32)
                    fb = lax.bitcast_convert_type(wrd, jnp.float32)
                    acc[g] = acc[g] + wsp * fa
                    acc[g + 2] = acc[g + 2] + wsp * fb
            bag = j * CB + k
            for c in range(D // 16):
                out_v[bag, pl.ds(c * 16, 16)] = acc[c]

    @pl.loop(0, NCH // NBUF)
    def _ring(i):
        j0 = i * NBUF
        for b in range(NBUF):
            j = j0 + b
            nxt = j + NBUF - 1
            pb = (b - 1) % NBUF

            @pl.when(nxt < NCH)
            def _():
                pltpu.async_copy(
                    table_hbm.at[idx_v.at[nxt]], rows_v.at[pb], sems[pb])

            pltpu.make_async_copy(
                table_hbm.at[idx_v.at[j]], rows_v.at[b], sems[b]).wait()
            compute_chunk(j, b)

    pltpu.sync_copy(out_v, out_hbm.at[wid])


def kernel(indices, weights, embeddings):
    idx3 = _remap_indices(indices.astype(jnp.int32)).reshape(NW, NCH, CI)
    w_pad = jnp.pad(weights, ((0, 0), (0, 64 - L)))
    w3 = w_pad.reshape(NW, NCH, CB * 64)
    table2 = _table_to_rowmajor(embeddings.T)
    out = _embedding_bag_sc(table2.reshape(V_PAD, D // 2), idx3, w3)
    return out.reshape(B, D)


# speedup vs baseline: 1.0692x; 1.0024x over previous
import jax
import jax.numpy as jnp
from jax import lax
from jax.experimental import pallas as pl
from jax.experimental.pallas import tpu as pltpu
from jax.experimental.pallas import tpu_sc as plsc
import functools

B = 16384
L = 50
D = 64
NW = 32
BW = B // NW
CB = 2
CI = CB * L
NCH = BW // CB
NBUF = 4

V = 1000000
TBLK = 16384
TGRID = (V + TBLK - 1) // TBLK
V_PAD = TGRID * TBLK
H = TBLK // 2


Q = TBLK // 4


def _transpose_body(tt_ref, out_ref):
    x = tt_ref[...]
    u = lax.bitcast_convert_type(x, jnp.uint32)
    r = u + jnp.uint32(0x8000)
    hw = r >> 16
    w = hw[0:D // 2, :] | (hw[D // 2:D, :] << 16)
    wi = lax.bitcast_convert_type(w, jnp.int32)
    out_ref[...] = jnp.concatenate(
        [wi[:, q * Q:(q + 1) * Q].T for q in range(4)], axis=1)


_table_to_rowmajor = pl.pallas_call(
    _transpose_body,
    grid=(TGRID,),
    in_specs=[pl.BlockSpec((D, TBLK), lambda j: (0, j))],
    out_specs=pl.BlockSpec((Q, 128), lambda j: (j, 0)),
    out_shape=jax.ShapeDtypeStruct((V_PAD // 4, 128), jnp.int32),
)


def _remap_indices(i):
    q = (i & (TBLK - 1)) >> 12
    a = i & (Q - 1)
    return (i & ~(TBLK - 1)) + (a << 2) + q


def _lane_splat(vec, idx):
    return lax.gather(
        vec, idx,
        dimension_numbers=lax.GatherDimensionNumbers(
            offset_dims=(), collapsed_slice_dims=(0,), start_index_map=(0,)),
        slice_sizes=(1,),
        mode=lax.GatherScatterMode.PROMISE_IN_BOUNDS)


_mesh = plsc.VectorSubcoreMesh(
    core_axis_name="c", subcore_axis_name="s", num_cores=2, num_subcores=16
)


@functools.partial(
    pl.kernel,
    out_type=jax.ShapeDtypeStruct((NW, BW, D), jnp.float32),
    mesh=_mesh,
    compiler_params=pltpu.CompilerParams(use_tc_tiling_on_sc=False),
    scratch_types=[
        pltpu.VMEM((NCH, CI), jnp.int32),
        pltpu.VMEM((NCH, CI), jnp.float32),
        pltpu.VMEM((NBUF, CI, D // 2), jnp.int32),
        pltpu.VMEM((BW, D), jnp.float32),
    ] + [pltpu.SemaphoreType.DMA] * NBUF,
)
def _embedding_bag_sc(table_hbm, idx_hbm, w_hbm, out_hbm,
                      idx_v, w_v, rows_v, out_v, *sems):
    wid = lax.axis_index("s") * 2 + lax.axis_index("c")
    pltpu.sync_copy(idx_hbm.at[wid], idx_v)
    pltpu.sync_copy(w_hbm.at[wid], w_v)

    for b in range(NBUF - 1):
        pltpu.async_copy(table_hbm.at[idx_v.at[b]], rows_v.at[b], sems[b])

    lane_idx = [jnp.full((16, 1), m, jnp.int32) for m in range(16)]

    def compute_chunk(j, b):
        for k in range(CB):
            base = k * L
            wv = [w_v[j, pl.ds(base + o, 16)] for o in (0, 16, 32, 34)]
            acc = [jnp.zeros((16,), jnp.float32) for _ in range(D // 16)]
            for l in range(L):
                r = k * L + l
                g, m = (l // 16, l % 16) if l < 48 else (3, l - 34)
                wsp = _lane_splat(wv[g], lane_idx[m])
                for g in range(2):
                    wrd = rows_v[b, r, pl.ds(g * 16, 16)]
                    fa = lax.bitcast_convert_type(wrd << 16, jnp.float32)
                    fb = lax.bitcast_convert_type(wrd, jnp.float32)
                    acc[g] = acc[g] + wsp * fa
                    acc[g + 2] = acc[g + 2] + wsp * fb
            bag = j * CB + k
            for c in range(D // 16):
                out_v[bag, pl.ds(c * 16, 16)] = acc[c]

    @pl.loop(0, NCH // NBUF)
    def _ring(i):
        j0 = i * NBUF
        for b in range(NBUF):
            j = j0 + b
            nxt = j + NBUF - 1
            pb = (b - 1) % NBUF

            @pl.when(nxt < NCH)
            def _():
                pltpu.async_copy(
                    table_hbm.at[idx_v.at[nxt]], rows_v.at[pb], sems[pb])

            pltpu.make_async_copy(
                table_hbm.at[idx_v.at[j]], rows_v.at[b], sems[b]).wait()
            compute_chunk(j, b)

    pltpu.sync_copy(out_v, out_hbm.at[wid])


def kernel(indices, weights, embeddings):
    idx3 = _remap_indices(indices.astype(jnp.int32)).reshape(NW, NCH, CI)
    w3 = weights.reshape(NW, NCH, CI)
    table2 = _table_to_rowmajor(embeddings.T)
    out = _embedding_bag_sc(table2.reshape(V_PAD, D // 2), idx3, w3)
    return out.reshape(B, D)
